# baseline (speedup 1.0000x reference)
import functools

import jax
import jax.numpy as jnp
from jax import lax
from jax.experimental import pallas as pl
from jax.experimental.pallas import tpu as pltpu
from jax.experimental.pallas import tpu_sc as plsc

_W = 16384
_CW = 256
_NC = _W // _CW

_K0 = 0
_K1 = 42
_K2 = _K0 ^ _K1 ^ 0x1BD11BDA

_ROT0 = (13, 15, 26, 6)
_ROT1 = (17, 29, 16, 24)

_TINY = 1.1754943508222875e-38
_LN2 = 0.6931471805599453

_SC_CH = 4096
_SC_SHARDS = 4
_SC_Q = 12
_SC_COLS = _SC_SHARDS * _SC_Q * _SC_CH
_V_REAL = 1_000_000
_B_REAL = 64
_LAST_BLOCK = (_V_REAL - 1) // _W
_TC_FULL = (_LAST_BLOCK * _W - _SC_COLS) // _W
_SC_C0 = _TC_FULL * _W

_LOG2_POLY = (
    0.01477872076596402,
    -0.07684872596702667,
    0.1904208313925399,
    -0.32311593513059617,
    0.47249952519075655,
    -0.7203866119437613,
    1.4426521110421746,
    3.1969782900697245e-07,
)


def _threefry_bits(cnt):
    ks = (_K0, _K1, _K2)
    x1 = cnt + jnp.uint32(_K1)
    x0 = x1
    first = True
    for g in range(1, 6):
        for r in _ROT0 if g % 2 == 1 else _ROT1:
            if first:
                first = False
            else:
                x0 = x0 + x1
            x1 = (x1 << jnp.uint32(r)) | (x1 >> jnp.uint32(32 - r))
            x1 = x1 ^ x0
        x0 = x0 + jnp.uint32(ks[g % 3])
        x1 = x1 + jnp.uint32((ks[(g + 1) % 3] + g) & 0xFFFFFFFF)
    return x0 ^ x1


def _uniform_from_bits(bits):
    fb = (bits >> jnp.uint32(9)) | jnp.uint32(0x3F800000)
    floats = lax.bitcast_convert_type(fb, jnp.float32) - jnp.float32(1.0)
    return jnp.maximum(floats, jnp.float32(_TINY))


def _neg_log_neg_log(bits):
    u = _uniform_from_bits(bits)
    return jnp.log(-jnp.log(u))


def _tc_kernel(x_ref, oi_ref, ov_ref, accv_ref, accc_ref, *, n_rows, n_vocab,
               grid, last_block):
    i = pl.program_id(0)
    shape = (n_rows, _CW)
    row = jax.lax.broadcasted_iota(jnp.int32, shape, 0)
    lane = jax.lax.broadcasted_iota(jnp.int32, shape, 1)
    base_cnt = row * n_vocab + lane
    row_end = row * n_vocab + n_vocab

    def chunk_body(c, carry, col0, masked):
        av, ac = carry
        cnt = base_cnt + (col0 + c * _CW)
        bits = _threefry_bits(cnt.astype(jnp.uint32))
        val = x_ref[:, pl.ds(c * _CW, _CW)] - _neg_log_neg_log(bits)
        if masked:
            val = jnp.where(cnt < row_end, val, -jnp.inf)
        upd = val > av
        return jnp.where(upd, val, av), jnp.where(upd, cnt, ac)

    init = (
        jnp.full(shape, -jnp.inf, jnp.float32),
        jnp.zeros(shape, jnp.int32),
    )

    @pl.when(i < grid - 1)
    def _full_block():
        av, ac = jax.lax.fori_loop(
            0, _NC, lambda c, s: chunk_body(c, s, i * _W, False), init,
            unroll=4
        )

        @pl.when(i == 0)
        def _init():
            accv_ref[...] = av
            accc_ref[...] = ac

        @pl.when(i > 0)
        def _update():
            gv = accv_ref[...]
            upd = av > gv
            accv_ref[...] = jnp.where(upd, av, gv)
            accc_ref[...] = jnp.where(upd, ac, accc_ref[...])

    @pl.when(i == grid - 1)
    def _tail_block():
        tail_cols = n_vocab - last_block * _W
        n_tail = -(-tail_cols // _CW)
        av, ac = jax.lax.fori_loop(
            0, n_tail, lambda c, s: chunk_body(c, s, last_block * _W, True),
            init, unroll=1
        )
        if grid > 1:
            gv = accv_ref[...]
            upd = av > gv
            fv = jnp.where(upd, av, gv)
            fc = jnp.where(upd, ac, accc_ref[...])
        else:
            fv, fc = av, ac
        col = fc - row * n_vocab
        m = jnp.max(fv, axis=1, keepdims=True)
        oi_ref[...] = jnp.min(jnp.where(fv == m, col, jnp.int32(2**30)), axis=1)
        ov_ref[...] = m[:, 0]


def _tc_argmax(logits, n_full_blocks, last_block):
    n_rows, n_vocab = logits.shape
    grid = n_full_blocks + 1
    kfn = functools.partial(
        _tc_kernel, n_rows=n_rows, n_vocab=n_vocab, grid=grid,
        last_block=last_block,
    )
    return pl.pallas_call(
        kfn,
        grid=(grid,),
        in_specs=[
            pl.BlockSpec(
                (n_rows, _W),
                lambda i: (0, jnp.where(i == grid - 1, last_block, i)),
            )
        ],
        out_specs=[
            pl.BlockSpec((n_rows,), lambda i: (0,)),
            pl.BlockSpec((n_rows,), lambda i: (0,)),
        ],
        out_shape=[
            jax.ShapeDtypeStruct((n_rows,), jnp.int32),
            jax.ShapeDtypeStruct((n_rows,), jnp.float32),
        ],
        scratch_shapes=[
            pltpu.VMEM((n_rows, _CW), jnp.float32),
            pltpu.VMEM((n_rows, _CW), jnp.int32),
        ],
    )(logits)


def _sc_log(x):
    ib = lax.bitcast_convert_type(x, jnp.int32)
    e = ((ib >> 23) - 127).astype(jnp.float32)
    m = lax.bitcast_convert_type(
        (ib & jnp.int32(0x7FFFFF)) | jnp.int32(0x3F800000), jnp.float32
    )
    z = m - jnp.float32(1.0)
    p = jnp.float32(_LOG2_POLY[0])
    for c in _LOG2_POLY[1:]:
        p = p * z + jnp.float32(c)
    return (e + p) * jnp.float32(_LN2)


def _sc_neg_log_u(bits):
    u = _uniform_from_bits(bits)
    d = jnp.float32(1.0) - u
    p = jnp.float32(1.0 / 6.0)
    for c in (1.0 / 5.0, 1.0 / 4.0, 1.0 / 3.0, 1.0 / 2.0, 1.0):
        p = p * d + jnp.float32(c)
    return jnp.where(u >= jnp.float32(0.84), d * p, jnp.float32(0.0) - _sc_log(u))


def _sc_screen(logits):
    n_rows, n_vocab = logits.shape
    mesh = plsc.VectorSubcoreMesh(core_axis_name="c", subcore_axis_name="s")

    @functools.partial(
        pl.kernel,
        mesh=mesh,
        out_type=(
            jax.ShapeDtypeStruct((32, 8, 16), jnp.int32),
            jax.ShapeDtypeStruct((32, 8, 16), jnp.float32),
        ),
        scratch_types=[
            pltpu.VMEM((8, _SC_CH), jnp.float32),
            pltpu.VMEM((8, 16), jnp.int32),
            pltpu.VMEM((8, 16), jnp.float32),
            pltpu.SemaphoreType.DMA,
        ],
    )
    def k(x_hbm, outc_hbm, outl_hbm, buf, mcref, mlref, sem):
        wid = lax.axis_index("s") * 2 + lax.axis_index("c")
        r0 = 8 * (wid // _SC_SHARDS)
        shard = wid % _SC_SHARDS
        lanes = lax.iota(jnp.int32, 16)

        def chunk(c, st):
            colbase = _SC_C0 + (_SC_SHARDS * c + shard) * _SC_CH
            cp = pltpu.make_async_copy(
                x_hbm.at[pl.ds(r0, 8), pl.ds(colbase, _SC_CH)], buf, sem
            )
            cp.start()
            cp.wait()

            def vec(j, s16):
                off = colbase + j * 16
                new_q = []
                new_c = []
                new_l = []
                for r in range(8):
                    cnt = lanes + ((r0 + r) * n_vocab + off)
                    bits = _threefry_bits(cnt.astype(jnp.uint32))
                    t = _sc_neg_log_u(bits)
                    lv = buf[r, pl.ds(j * 16, 16)]
                    qv = t * jnp.exp(jnp.float32(0.0) - lv)
                    upd = qv < s16[r]
                    new_q.append(jnp.where(upd, qv, s16[r]))
                    new_c.append(jnp.where(upd, cnt, s16[8 + r]))
                    new_l.append(jnp.where(upd, lv, s16[16 + r]))
                return tuple(new_q) + tuple(new_c) + tuple(new_l)

            return lax.fori_loop(0, _SC_CH // 16, vec, st, unroll=2)

        inf = jnp.full((16,), jnp.inf, jnp.float32)
        zero = jnp.zeros((16,), jnp.int32)
        zf = jnp.zeros((16,), jnp.float32)
        st = lax.fori_loop(0, _SC_Q, chunk, (inf,) * 8 + (zero,) * 8 + (zf,) * 8)
        for r in range(8):
            mcref[r, :] = st[8 + r]
            mlref[r, :] = st[16 + r]
        cpc = pltpu.make_async_copy(mcref, outc_hbm.at[wid], sem)
        cpc.start()
        cpc.wait()
        cpl = pltpu.make_async_copy(mlref, outl_hbm.at[wid], sem)
        cpl.start()
        cpl.wait()

    return k(logits)


def _merge(logits, col_tc, val_tc, outc, outl):
    n_rows, n_vocab = logits.shape
    def _cands(o):
        return (
            o.reshape(8, _SC_SHARDS, 8, 16)
            .transpose(0, 2, 1, 3)
            .reshape(n_rows, 16 * _SC_SHARDS)
        )

    cand_c = _cands(outc)
    cand_l = _cands(outl)
    bits = _threefry_bits(cand_c.astype(jnp.uint32))
    g = _neg_log_neg_log(bits)
    w = cand_l - g
    m = jnp.max(w, axis=1)
    csel = jnp.min(
        jnp.where(w == m[:, None], cand_c, jnp.int32(2**31 - 1)), axis=1
    )
    col_sc = csel - jnp.arange(n_rows, dtype=jnp.int32) * n_vocab
    sel_sc = (m > val_tc) | ((m == val_tc) & (col_sc < col_tc))
    return jnp.where(sel_sc, col_sc, col_tc)


def kernel(logits):
    n_rows, n_vocab = logits.shape
    if n_rows == _B_REAL and n_vocab == _V_REAL:
        col_tc, val_tc = _tc_argmax(logits, _TC_FULL, _LAST_BLOCK)
        outc, outl = _sc_screen(logits)
        out = _merge(logits, col_tc, val_tc, outc, outl)
    else:
        grid = -(-n_vocab // _W)
        out, _ = _tc_argmax(logits, grid - 1, grid - 1)
    return out.astype(jnp.int64)


# speedup vs baseline: 1.0412x; 1.0412x over previous
import functools

import jax
import jax.numpy as jnp
from jax import lax
from jax.experimental import pallas as pl
from jax.experimental.pallas import tpu as pltpu
from jax.experimental.pallas import tpu_sc as plsc

_W = 16384
_CW = 128
_NC = _W // _CW

_K0 = 0
_K1 = 42
_K2 = _K0 ^ _K1 ^ 0x1BD11BDA

_ROT0 = (13, 15, 26, 6)
_ROT1 = (17, 29, 16, 24)

_TINY = 1.1754943508222875e-38
_LN2 = 0.6931471805599453

_SC_CH = 4096
_SC_SHARDS = 4
_SC_Q = 12
_SC_COLS = _SC_SHARDS * _SC_Q * _SC_CH
_V_REAL = 1_000_000
_B_REAL = 64
_LAST_BLOCK = (_V_REAL - 1) // _W
_TC_FULL = (_LAST_BLOCK * _W - _SC_COLS) // _W
_SC_C0 = _TC_FULL * _W

_LOG2_POLY = (
    0.01477872076596402,
    -0.07684872596702667,
    0.1904208313925399,
    -0.32311593513059617,
    0.47249952519075655,
    -0.7203866119437613,
    1.4426521110421746,
    3.1969782900697245e-07,
)


def _threefry_bits(cnt):
    ks = (_K0, _K1, _K2)
    x1 = cnt + jnp.uint32(_K1)
    x0 = x1
    first = True
    for g in range(1, 6):
        for r in _ROT0 if g % 2 == 1 else _ROT1:
            if first:
                first = False
            else:
                x0 = x0 + x1
            x1 = (x1 << jnp.uint32(r)) | (x1 >> jnp.uint32(32 - r))
            x1 = x1 ^ x0
        x0 = x0 + jnp.uint32(ks[g % 3])
        x1 = x1 + jnp.uint32((ks[(g + 1) % 3] + g) & 0xFFFFFFFF)
    return x0 ^ x1


def _uniform_from_bits(bits):
    fb = (bits >> jnp.uint32(9)) | jnp.uint32(0x3F800000)
    floats = lax.bitcast_convert_type(fb, jnp.float32) - jnp.float32(1.0)
    return jnp.maximum(floats, jnp.float32(_TINY))


def _neg_log_neg_log(bits):
    u = _uniform_from_bits(bits)
    return jnp.log(-jnp.log(u))


def _tc_kernel(x_ref, oi_ref, ov_ref, accv_ref, accc_ref, *, n_rows, n_vocab,
               grid, last_block):
    i = pl.program_id(0)
    shape = (n_rows, _CW)
    row = jax.lax.broadcasted_iota(jnp.int32, shape, 0)
    lane = jax.lax.broadcasted_iota(jnp.int32, shape, 1)
    base_cnt = row * n_vocab + lane
    row_end = row * n_vocab + n_vocab

    def chunk_body(c, carry, col0, masked):
        av, ac = carry
        cnt = base_cnt + (col0 + c * _CW)
        bits = _threefry_bits(cnt.astype(jnp.uint32))
        val = x_ref[:, pl.ds(c * _CW, _CW)] - _neg_log_neg_log(bits)
        if masked:
            val = jnp.where(cnt < row_end, val, -jnp.inf)
        upd = val > av
        return jnp.where(upd, val, av), jnp.where(upd, cnt, ac)

    init = (
        jnp.full(shape, -jnp.inf, jnp.float32),
        jnp.zeros(shape, jnp.int32),
    )

    @pl.when(i < grid - 1)
    def _full_block():
        av, ac = jax.lax.fori_loop(
            0, _NC, lambda c, s: chunk_body(c, s, i * _W, False), init,
            unroll=8
        )

        @pl.when(i == 0)
        def _init():
            accv_ref[...] = av
            accc_ref[...] = ac

        @pl.when(i > 0)
        def _update():
            gv = accv_ref[...]
            upd = av > gv
            accv_ref[...] = jnp.where(upd, av, gv)
            accc_ref[...] = jnp.where(upd, ac, accc_ref[...])

    @pl.when(i == grid - 1)
    def _tail_block():
        tail_cols = n_vocab - last_block * _W
        n_tail = -(-tail_cols // _CW)
        av, ac = jax.lax.fori_loop(
            0, n_tail, lambda c, s: chunk_body(c, s, last_block * _W, True),
            init, unroll=1
        )
        if grid > 1:
            gv = accv_ref[...]
            upd = av > gv
            fv = jnp.where(upd, av, gv)
            fc = jnp.where(upd, ac, accc_ref[...])
        else:
            fv, fc = av, ac
        col = fc - row * n_vocab
        m = jnp.max(fv, axis=1, keepdims=True)
        oi_ref[...] = jnp.min(jnp.where(fv == m, col, jnp.int32(2**30)), axis=1)
        ov_ref[...] = m[:, 0]


def _tc_argmax(logits, n_full_blocks, last_block):
    n_rows, n_vocab = logits.shape
    grid = n_full_blocks + 1
    kfn = functools.partial(
        _tc_kernel, n_rows=n_rows, n_vocab=n_vocab, grid=grid,
        last_block=last_block,
    )
    return pl.pallas_call(
        kfn,
        grid=(grid,),
        in_specs=[
            pl.BlockSpec(
                (n_rows, _W),
                lambda i: (0, jnp.where(i == grid - 1, last_block, i)),
            )
        ],
        out_specs=[
            pl.BlockSpec((n_rows,), lambda i: (0,)),
            pl.BlockSpec((n_rows,), lambda i: (0,)),
        ],
        out_shape=[
            jax.ShapeDtypeStruct((n_rows,), jnp.int32),
            jax.ShapeDtypeStruct((n_rows,), jnp.float32),
        ],
        scratch_shapes=[
            pltpu.VMEM((n_rows, _CW), jnp.float32),
            pltpu.VMEM((n_rows, _CW), jnp.int32),
        ],
    )(logits)


def _sc_log(x):
    ib = lax.bitcast_convert_type(x, jnp.int32)
    e = ((ib >> 23) - 127).astype(jnp.float32)
    m = lax.bitcast_convert_type(
        (ib & jnp.int32(0x7FFFFF)) | jnp.int32(0x3F800000), jnp.float32
    )
    z = m - jnp.float32(1.0)
    p = jnp.float32(_LOG2_POLY[0])
    for c in _LOG2_POLY[1:]:
        p = p * z + jnp.float32(c)
    return (e + p) * jnp.float32(_LN2)


def _sc_neg_log_u(bits):
    u = _uniform_from_bits(bits)
    d = jnp.float32(1.0) - u
    p = jnp.float32(1.0 / 6.0)
    for c in (1.0 / 5.0, 1.0 / 4.0, 1.0 / 3.0, 1.0 / 2.0, 1.0):
        p = p * d + jnp.float32(c)
    return jnp.where(u >= jnp.float32(0.84), d * p, jnp.float32(0.0) - _sc_log(u))


def _sc_screen(logits):
    n_rows, n_vocab = logits.shape
    mesh = plsc.VectorSubcoreMesh(core_axis_name="c", subcore_axis_name="s")

    @functools.partial(
        pl.kernel,
        mesh=mesh,
        out_type=(
            jax.ShapeDtypeStruct((32, 8, 16), jnp.int32),
            jax.ShapeDtypeStruct((32, 8, 16), jnp.float32),
        ),
        scratch_types=[
            pltpu.VMEM((8, _SC_CH), jnp.float32),
            pltpu.VMEM((8, 16), jnp.int32),
            pltpu.VMEM((8, 16), jnp.float32),
            pltpu.SemaphoreType.DMA,
        ],
    )
    def k(x_hbm, outc_hbm, outl_hbm, buf, mcref, mlref, sem):
        wid = lax.axis_index("s") * 2 + lax.axis_index("c")
        r0 = 8 * (wid // _SC_SHARDS)
        shard = wid % _SC_SHARDS
        lanes = lax.iota(jnp.int32, 16)

        def chunk(c, st):
            colbase = _SC_C0 + (_SC_SHARDS * c + shard) * _SC_CH
            cp = pltpu.make_async_copy(
                x_hbm.at[pl.ds(r0, 8), pl.ds(colbase, _SC_CH)], buf, sem
            )
            cp.start()
            cp.wait()

            def vec(j, s16):
                off = colbase + j * 16
                new_q = []
                new_c = []
                new_l = []
                for r in range(8):
                    cnt = lanes + ((r0 + r) * n_vocab + off)
                    bits = _threefry_bits(cnt.astype(jnp.uint32))
                    t = _sc_neg_log_u(bits)
                    lv = buf[r, pl.ds(j * 16, 16)]
                    qv = t * jnp.exp(jnp.float32(0.0) - lv)
                    upd = qv < s16[r]
                    new_q.append(jnp.where(upd, qv, s16[r]))
                    new_c.append(jnp.where(upd, cnt, s16[8 + r]))
                    new_l.append(jnp.where(upd, lv, s16[16 + r]))
                return tuple(new_q) + tuple(new_c) + tuple(new_l)

            return lax.fori_loop(0, _SC_CH // 16, vec, st, unroll=2)

        inf = jnp.full((16,), jnp.inf, jnp.float32)
        zero = jnp.zeros((16,), jnp.int32)
        zf = jnp.zeros((16,), jnp.float32)
        st = lax.fori_loop(0, _SC_Q, chunk, (inf,) * 8 + (zero,) * 8 + (zf,) * 8)
        for r in range(8):
            mcref[r, :] = st[8 + r]
            mlref[r, :] = st[16 + r]
        cpc = pltpu.make_async_copy(mcref, outc_hbm.at[wid], sem)
        cpc.start()
        cpc.wait()
        cpl = pltpu.make_async_copy(mlref, outl_hbm.at[wid], sem)
        cpl.start()
        cpl.wait()

    return k(logits)


def _merge(logits, col_tc, val_tc, outc, outl):
    n_rows, n_vocab = logits.shape
    def _cands(o):
        return (
            o.reshape(8, _SC_SHARDS, 8, 16)
            .transpose(0, 2, 1, 3)
            .reshape(n_rows, 16 * _SC_SHARDS)
        )

    cand_c = _cands(outc)
    cand_l = _cands(outl)
    bits = _threefry_bits(cand_c.astype(jnp.uint32))
    g = _neg_log_neg_log(bits)
    w = cand_l - g
    m = jnp.max(w, axis=1)
    csel = jnp.min(
        jnp.where(w == m[:, None], cand_c, jnp.int32(2**31 - 1)), axis=1
    )
    col_sc = csel - jnp.arange(n_rows, dtype=jnp.int32) * n_vocab
    sel_sc = (m > val_tc) | ((m == val_tc) & (col_sc < col_tc))
    return jnp.where(sel_sc, col_sc, col_tc)


def kernel(logits):
    n_rows, n_vocab = logits.shape
    if n_rows == _B_REAL and n_vocab == _V_REAL:
        col_tc, val_tc = _tc_argmax(logits, _TC_FULL, _LAST_BLOCK)
        outc, outl = _sc_screen(logits)
        out = _merge(logits, col_tc, val_tc, outc, outl)
    else:
        grid = -(-n_vocab // _W)
        out, _ = _tc_argmax(logits, grid - 1, grid - 1)
    return out.astype(jnp.int64)
